# baseline (speedup 1.0000x reference)
import functools

import jax
import jax.numpy as jnp
from jax import lax
from jax.experimental import pallas as pl
from jax.experimental.pallas import tpu as pltpu
from jax.experimental.pallas import tpu_sc as plsc

_K = 8192
_D = 32
_N = 4096
_NT = 512
_KC = 2048
_COMMIT = 0.25


def _argmin_body(x_ref, emb_ref, idx_ref, loss_ref):
    i = pl.program_id(0)
    x = x_ref[:, :]
    xsq = jnp.sum(x * x, axis=1, keepdims=True)

    run_min = jnp.full((_NT, 1), jnp.inf, jnp.float32)
    run_arg = jnp.zeros((_NT, 1), jnp.int32)
    for c in range(_K // _KC):
        e = emb_ref[pl.ds(c * _KC, _KC), :]
        esq = jnp.sum(e * e, axis=1, keepdims=True)
        cross = lax.dot_general(
            x, e, (((1,), (1,)), ((), ())),
            preferred_element_type=jnp.float32)
        dist = (xsq - 2.0 * cross + esq.T) / float(_D)
        cmin = jnp.min(dist, axis=1, keepdims=True)
        iota = lax.broadcasted_iota(jnp.int32, (_NT, _KC), 1) + c * _KC
        carg = jnp.min(jnp.where(dist == cmin, iota, jnp.int32(2**30)),
                       axis=1, keepdims=True)
        better = cmin < run_min
        run_arg = jnp.where(better, carg, run_arg)
        run_min = jnp.where(better, cmin, run_min)

    idx_ref[0, :, :] = run_arg

    @pl.when(i == 0)
    def _():
        loss_ref[0, 0] = 0.0

    loss_ref[0, 0] += jnp.sum(run_min)

    @pl.when(i == pl.num_programs(0) - 1)
    def _():
        loss_ref[0, 0] = loss_ref[0, 0] * ((1.0 + _COMMIT) / _N)


def _tc_argmin(flat_x, emb):
    grid = (_N // _NT,)
    idx, loss = pl.pallas_call(
        _argmin_body,
        grid=grid,
        in_specs=[
            pl.BlockSpec((_NT, _D), lambda i: (i, 0)),
            pl.BlockSpec((_K, _D), lambda i: (0, 0)),
        ],
        out_specs=[
            pl.BlockSpec((1, _NT, 1), lambda i: (i, 0, 0)),
            pl.BlockSpec(memory_space=pltpu.MemorySpace.SMEM),
        ],
        out_shape=[
            jax.ShapeDtypeStruct((grid[0], _NT, 1), jnp.int32),
            jax.ShapeDtypeStruct((1, 1), jnp.float32),
        ],
    )(flat_x, emb)
    return idx.reshape(_N), loss[0, 0]


_DP = 128


def _sc_gather(emb_pad, idx):
    info = plsc.get_sparse_core_info()
    nw = info.num_cores * info.num_subcores
    bpw = _N // nw
    mesh = plsc.VectorSubcoreMesh(core_axis_name="c", subcore_axis_name="s")

    @functools.partial(
        pl.kernel, mesh=mesh,
        out_type=jax.ShapeDtypeStruct((_N, _DP), jnp.float32),
        scratch_types=[
            pltpu.VMEM((bpw,), jnp.int32),
            pltpu.VMEM((bpw, _DP), jnp.float32),
            pltpu.SemaphoreType.DMA,
        ],
    )
    def gather(table_hbm, idx_hbm, out_hbm, idx_v, rows_v, sem):
        wid = lax.axis_index("s") * info.num_cores + lax.axis_index("c")
        base = wid * bpw
        pltpu.sync_copy(idx_hbm.at[pl.ds(base, bpw)], idx_v)
        pltpu.async_copy(table_hbm.at[idx_v], rows_v, sem).wait()
        pltpu.sync_copy(rows_v, out_hbm.at[pl.ds(base, bpw)])

    return gather(emb_pad, idx)


def kernel(x, embedding_weight):
    b, c, h, w = x.shape
    flat_x = jnp.transpose(x, (0, 2, 3, 1)).reshape(_N, _D)
    idx, loss = _tc_argmin(flat_x, embedding_weight)
    emb_pad = jnp.pad(embedding_weight, ((0, 0), (0, _DP - _D)))
    quant_flat = _sc_gather(emb_pad, idx)[:, :_D]
    quantized_out = jnp.transpose(
        quant_flat.reshape(b, h, w, c), (0, 3, 1, 2))
    indices_out = idx.reshape(b, h * w)
    return (loss, quantized_out, indices_out)


# speedup vs baseline: 1.5566x; 1.5566x over previous
import functools

import jax
import jax.numpy as jnp
from jax import lax
from jax.experimental import pallas as pl
from jax.experimental.pallas import tpu as pltpu
from jax.experimental.pallas import tpu_sc as plsc

_K = 8192
_D = 32
_N = 4096
_NT = 512
_KC = 2048
_COMMIT = 0.25


_DA = _D + 2


def _argmin_body(x_ref, emb_ref, idx_ref, loss_ref, xa_ref, ea_ref, io_ref):
    i = pl.program_id(0)

    @pl.when(i == 0)
    def _():
        e = emb_ref[:, :]
        ea_ref[:, 0:_D] = e
        ea_ref[:, _D:_D + 1] = jnp.ones((_K, 1), jnp.float32)
        ea_ref[:, _D + 1:_DA] = jnp.sum(e * e, axis=1, keepdims=True)
        io_ref[0:1, :] = lax.broadcasted_iota(
            jnp.int32, (1, _K), 1).astype(jnp.float32)

    x = x_ref[:, :]
    xa_ref[:, 0:_D] = -2.0 * x
    xa_ref[:, _D:_D + 1] = jnp.sum(x * x, axis=1, keepdims=True)
    xa_ref[:, _D + 1:_DA] = jnp.ones((_NT, 1), jnp.float32)
    xa = xa_ref[:, :]

    run_min = jnp.full((_NT, 1), jnp.inf, jnp.float32)
    run_arg = jnp.full((_NT, 1), 0.0, jnp.float32)
    for c in range(_K // _KC):
        raw = lax.dot_general(
            xa, ea_ref[pl.ds(c * _KC, _KC), :], (((1,), (1,)), ((), ())),
            preferred_element_type=jnp.float32)
        cmin = jnp.min(raw, axis=1, keepdims=True)
        carg = jnp.min(jnp.where(raw == cmin, io_ref[0:1, pl.ds(c * _KC, _KC)],
                                 jnp.float32(1e9)),
                       axis=1, keepdims=True)
        better = cmin < run_min
        run_arg = jnp.where(better, carg, run_arg)
        run_min = jnp.where(better, cmin, run_min)

    idx_ref[0, :, :] = run_arg.astype(jnp.int32)

    @pl.when(i == 0)
    def _():
        loss_ref[0, 0] = 0.0

    loss_ref[0, 0] += jnp.sum(run_min)

    @pl.when(i == pl.num_programs(0) - 1)
    def _():
        loss_ref[0, 0] = loss_ref[0, 0] * ((1.0 + _COMMIT) / (_N * _D))


def _tc_argmin(flat_x, emb):
    grid = (_N // _NT,)
    idx, loss = pl.pallas_call(
        _argmin_body,
        grid=grid,
        in_specs=[
            pl.BlockSpec((_NT, _D), lambda i: (i, 0)),
            pl.BlockSpec((_K, _D), lambda i: (0, 0)),
        ],
        out_specs=[
            pl.BlockSpec((1, _NT, 1), lambda i: (i, 0, 0)),
            pl.BlockSpec(memory_space=pltpu.MemorySpace.SMEM),
        ],
        out_shape=[
            jax.ShapeDtypeStruct((grid[0], _NT, 1), jnp.int32),
            jax.ShapeDtypeStruct((1, 1), jnp.float32),
        ],
        scratch_shapes=[
            pltpu.VMEM((_NT, _DA), jnp.float32),
            pltpu.VMEM((_K, _DA), jnp.float32),
            pltpu.VMEM((1, _K), jnp.float32),
        ],
    )(flat_x, emb)
    return idx.reshape(_N), loss[0, 0]


_DP = 128


def _sc_gather(emb_pad, idx):
    info = plsc.get_sparse_core_info()
    nw = info.num_cores * info.num_subcores
    bpw = _N // nw
    mesh = plsc.VectorSubcoreMesh(core_axis_name="c", subcore_axis_name="s")

    @functools.partial(
        pl.kernel, mesh=mesh,
        out_type=jax.ShapeDtypeStruct((_N, _DP), jnp.float32),
        scratch_types=[
            pltpu.VMEM((bpw,), jnp.int32),
            pltpu.VMEM((bpw, _DP), jnp.float32),
            pltpu.SemaphoreType.DMA,
        ],
    )
    def gather(table_hbm, idx_hbm, out_hbm, idx_v, rows_v, sem):
        wid = lax.axis_index("s") * info.num_cores + lax.axis_index("c")
        base = wid * bpw
        pltpu.sync_copy(idx_hbm.at[pl.ds(base, bpw)], idx_v)
        pltpu.async_copy(table_hbm.at[idx_v], rows_v, sem).wait()
        pltpu.sync_copy(rows_v, out_hbm.at[pl.ds(base, bpw)])

    return gather(emb_pad, idx)


def kernel(x, embedding_weight):
    b, c, h, w = x.shape
    flat_x = jnp.transpose(x, (0, 2, 3, 1)).reshape(_N, _D)
    idx, loss = _tc_argmin(flat_x, embedding_weight)
    emb_pad = jnp.pad(embedding_weight, ((0, 0), (0, _DP - _D)))
    quant_flat = _sc_gather(emb_pad, idx)[:, :_D]
    quantized_out = jnp.transpose(
        quant_flat.reshape(b, h, w, c), (0, 3, 1, 2))
    indices_out = idx.reshape(b, h * w)
    return (loss, quantized_out, indices_out)


# speedup vs baseline: 1.6170x; 1.0388x over previous
import functools

import jax
import jax.numpy as jnp
from jax import lax
from jax.experimental import pallas as pl
from jax.experimental.pallas import tpu as pltpu
from jax.experimental.pallas import tpu_sc as plsc

_K = 8192
_D = 32
_N = 4096
_NT = 512
_KC = 2048
_DA = _D + 2
_DP = 128
_COMMIT = 0.25


def _argmin_body(x_ref, emb_ref, idx_ref, loss_ref, pad_ref,
                 xa_ref, ea_ref, io_ref):
    i = pl.program_id(0)

    @pl.when(i == 0)
    def _():
        e = emb_ref[:, :]
        ea_ref[:, 0:_D] = e
        ea_ref[:, _D:_D + 1] = jnp.ones((_K, 1), jnp.float32)
        ea_ref[:, _D + 1:_DA] = jnp.sum(e * e, axis=1, keepdims=True)
        pad_ref[:, :] = jnp.zeros((_K, _DP), jnp.float32)
        pad_ref[:, 0:_D] = e
        io_ref[:, :] = lax.broadcasted_iota(
            jnp.int32, (_KC, _NT), 0).astype(jnp.float32)

    x = x_ref[0, :, :]
    xa_ref[0:_D, :] = -2.0 * x
    xa_ref[_D:_D + 1, :] = jnp.sum(x * x, axis=0, keepdims=True)
    xa_ref[_D + 1:_DA, :] = jnp.ones((1, _NT), jnp.float32)
    xa = xa_ref[:, :]

    run_min = jnp.full((1, _NT), jnp.inf, jnp.float32)
    run_arg = jnp.full((1, _NT), 0.0, jnp.float32)
    for c in range(_K // _KC):
        raw = lax.dot_general(
            ea_ref[pl.ds(c * _KC, _KC), :], xa, (((1,), (0,)), ((), ())),
            preferred_element_type=jnp.float32)
        cmin = jnp.min(raw, axis=0, keepdims=True)
        carg = jnp.min(jnp.where(raw == cmin, io_ref[:, :], jnp.float32(1e9)),
                       axis=0, keepdims=True)
        if c:
            carg = carg + jnp.float32(c * _KC)
            better = cmin < run_min
            run_arg = jnp.where(better, carg, run_arg)
            run_min = jnp.where(better, cmin, run_min)
        else:
            run_arg, run_min = carg, cmin

    idx_ref[0, :, :] = run_arg.astype(jnp.int32)

    @pl.when(i == 0)
    def _():
        loss_ref[0, 0] = 0.0

    loss_ref[0, 0] += jnp.sum(run_min)

    @pl.when(i == pl.num_programs(0) - 1)
    def _():
        loss_ref[0, 0] = loss_ref[0, 0] * ((1.0 + _COMMIT) / (_N * _D))


def _tc_argmin(x3, emb):
    grid = (_N // _NT,)
    idx, loss, emb_pad = pl.pallas_call(
        _argmin_body,
        grid=grid,
        in_specs=[
            pl.BlockSpec((1, _D, _NT), lambda i: (i // 2, 0, i % 2)),
            pl.BlockSpec((_K, _D), lambda i: (0, 0)),
        ],
        out_specs=[
            pl.BlockSpec((1, 1, _NT), lambda i: (i, 0, 0)),
            pl.BlockSpec(memory_space=pltpu.MemorySpace.SMEM),
            pl.BlockSpec((_K, _DP), lambda i: (0, 0)),
        ],
        out_shape=[
            jax.ShapeDtypeStruct((grid[0], 1, _NT), jnp.int32),
            jax.ShapeDtypeStruct((1, 1), jnp.float32),
            jax.ShapeDtypeStruct((_K, _DP), jnp.float32),
        ],
        scratch_shapes=[
            pltpu.VMEM((_DA, _NT), jnp.float32),
            pltpu.VMEM((_K, _DA), jnp.float32),
            pltpu.VMEM((_KC, _NT), jnp.float32),
        ],
    )(x3, emb)
    return idx.reshape(_N), loss[0, 0], emb_pad


def _sc_gather(emb_pad, idx):
    info = plsc.get_sparse_core_info()
    nw = info.num_cores * info.num_subcores
    bpw = _N // nw
    mesh = plsc.VectorSubcoreMesh(core_axis_name="c", subcore_axis_name="s")

    @functools.partial(
        pl.kernel, mesh=mesh,
        out_type=jax.ShapeDtypeStruct((_N, _DP), jnp.float32),
        scratch_types=[
            pltpu.VMEM((bpw,), jnp.int32),
            pltpu.VMEM((bpw, _DP), jnp.float32),
            pltpu.SemaphoreType.DMA,
        ],
    )
    def gather(table_hbm, idx_hbm, out_hbm, idx_v, rows_v, sem):
        wid = lax.axis_index("s") * info.num_cores + lax.axis_index("c")
        base = wid * bpw
        pltpu.sync_copy(idx_hbm.at[pl.ds(base, bpw)], idx_v)
        pltpu.async_copy(table_hbm.at[idx_v], rows_v, sem).wait()
        pltpu.sync_copy(rows_v, out_hbm.at[pl.ds(base, bpw)])

    return gather(emb_pad, idx)


def kernel(x, embedding_weight):
    b, c, h, w = x.shape
    x3 = x.reshape(b, c, h * w)
    idx, loss, emb_pad = _tc_argmin(x3, embedding_weight)
    quant_flat = _sc_gather(emb_pad, idx)[:, :_D]
    quantized_out = jnp.transpose(
        quant_flat.reshape(b, h, w, c), (0, 3, 1, 2))
    indices_out = idx.reshape(b, h * w)
    return (loss, quantized_out, indices_out)


# speedup vs baseline: 1.6768x; 1.0370x over previous
import functools

import jax
import jax.numpy as jnp
from jax import lax
from jax.experimental import pallas as pl
from jax.experimental.pallas import tpu as pltpu
from jax.experimental.pallas import tpu_sc as plsc

_K = 8192
_D = 32
_N = 4096
_NT = 1024
_KC = 2048
_DA = _D + 2
_DP = 128
_COMMIT = 0.25


def _argmin_body(x_ref, emb_ref, idx_ref, loss_ref, pad_ref,
                 xa_ref, ea_ref, io_ref):
    i = pl.program_id(0)

    @pl.when(i == 0)
    def _():
        e = emb_ref[:, :]
        ea_ref[:, 0:_D] = e
        ea_ref[:, _D:_D + 1] = jnp.ones((_K, 1), jnp.float32)
        ea_ref[:, _D + 1:_DA] = jnp.sum(e * e, axis=1, keepdims=True)
        pad_ref[:, :] = jnp.zeros((_K, _DP), jnp.float32)
        pad_ref[:, 0:_D] = e
        io_ref[:, :] = lax.broadcasted_iota(
            jnp.int32, (_KC, _NT), 0).astype(jnp.float32)

    x = x_ref[0, :, :]
    xa_ref[0:_D, :] = -2.0 * x
    xa_ref[_D:_D + 1, :] = jnp.sum(x * x, axis=0, keepdims=True)
    xa_ref[_D + 1:_DA, :] = jnp.ones((1, _NT), jnp.float32)
    xa = xa_ref[:, :]

    run_min = jnp.full((1, _NT), jnp.inf, jnp.float32)
    run_arg = jnp.full((1, _NT), 0.0, jnp.float32)
    for c in range(_K // _KC):
        raw = lax.dot_general(
            ea_ref[pl.ds(c * _KC, _KC), :], xa, (((1,), (0,)), ((), ())),
            preferred_element_type=jnp.float32)
        cmin = jnp.min(raw, axis=0, keepdims=True)
        carg = jnp.min(jnp.where(raw == cmin, io_ref[:, :], jnp.float32(1e9)),
                       axis=0, keepdims=True)
        if c:
            carg = carg + jnp.float32(c * _KC)
            better = cmin < run_min
            run_arg = jnp.where(better, carg, run_arg)
            run_min = jnp.where(better, cmin, run_min)
        else:
            run_arg, run_min = carg, cmin

    idx_ref[0, :, :] = run_arg.astype(jnp.int32)

    @pl.when(i == 0)
    def _():
        loss_ref[0, 0] = 0.0

    loss_ref[0, 0] += jnp.sum(run_min)

    @pl.when(i == pl.num_programs(0) - 1)
    def _():
        loss_ref[0, 0] = loss_ref[0, 0] * ((1.0 + _COMMIT) / (_N * _D))


def _tc_argmin(x3, emb):
    grid = (_N // _NT,)
    idx, loss, emb_pad = pl.pallas_call(
        _argmin_body,
        grid=grid,
        in_specs=[
            pl.BlockSpec((1, _D, _NT), lambda i: (i, 0, 0)),
            pl.BlockSpec((_K, _D), lambda i: (0, 0)),
        ],
        out_specs=[
            pl.BlockSpec((1, 1, _NT), lambda i: (i, 0, 0)),
            pl.BlockSpec(memory_space=pltpu.MemorySpace.SMEM),
            pl.BlockSpec((_K, _DP), lambda i: (0, 0)),
        ],
        out_shape=[
            jax.ShapeDtypeStruct((grid[0], 1, _NT), jnp.int32),
            jax.ShapeDtypeStruct((1, 1), jnp.float32),
            jax.ShapeDtypeStruct((_K, _DP), jnp.float32),
        ],
        scratch_shapes=[
            pltpu.VMEM((_DA, _NT), jnp.float32),
            pltpu.VMEM((_K, _DA), jnp.float32),
            pltpu.VMEM((_KC, _NT), jnp.float32),
        ],
    )(x3, emb)
    return idx.reshape(_N), loss[0, 0], emb_pad


def _sc_gather(emb_pad, idx):
    info = plsc.get_sparse_core_info()
    nw = info.num_cores * info.num_subcores
    bpw = _N // nw
    mesh = plsc.VectorSubcoreMesh(core_axis_name="c", subcore_axis_name="s")

    @functools.partial(
        pl.kernel, mesh=mesh,
        out_type=jax.ShapeDtypeStruct((_N, _DP), jnp.float32),
        scratch_types=[
            pltpu.VMEM((bpw,), jnp.int32),
            pltpu.VMEM((bpw, _DP), jnp.float32),
            pltpu.SemaphoreType.DMA,
        ],
    )
    def gather(table_hbm, idx_hbm, out_hbm, idx_v, rows_v, sem):
        wid = lax.axis_index("s") * info.num_cores + lax.axis_index("c")
        base = wid * bpw
        pltpu.sync_copy(idx_hbm.at[pl.ds(base, bpw)], idx_v)
        pltpu.async_copy(table_hbm.at[idx_v], rows_v, sem).wait()
        pltpu.sync_copy(rows_v, out_hbm.at[pl.ds(base, bpw)])

    return gather(emb_pad, idx)


def kernel(x, embedding_weight):
    b, c, h, w = x.shape
    x3 = x.reshape(b, c, h * w)
    idx, loss, emb_pad = _tc_argmin(x3, embedding_weight)
    quant_flat = _sc_gather(emb_pad, idx)[:, :_D]
    quantized_out = jnp.transpose(
        quant_flat.reshape(b, h, w, c), (0, 3, 1, 2))
    indices_out = idx.reshape(b, h * w)
    return (loss, quantized_out, indices_out)


# speedup vs baseline: 1.7012x; 1.0145x over previous
import functools

import jax
import jax.numpy as jnp
from jax import lax
from jax.experimental import pallas as pl
from jax.experimental.pallas import tpu as pltpu
from jax.experimental.pallas import tpu_sc as plsc

_K = 8192
_D = 32
_N = 4096
_NT = 1024
_KC = 1024
_DA = _D + 2
_DP = 128
_COMMIT = 0.25


def _argmin_body(x_ref, emb_ref, idx_ref, loss_ref, pad_ref,
                 xa_ref, ea_ref, io_ref):
    i = pl.program_id(0)

    @pl.when(i == 0)
    def _():
        e = emb_ref[:, :]
        ea_ref[:, 0:_D] = e
        ea_ref[:, _D:_D + 1] = jnp.ones((_K, 1), jnp.float32)
        ea_ref[:, _D + 1:_DA] = jnp.sum(e * e, axis=1, keepdims=True)
        pad_ref[:, :] = jnp.zeros((_K, _DP), jnp.float32)
        pad_ref[:, 0:_D] = e
        io_ref[:, :] = lax.broadcasted_iota(
            jnp.int32, (_KC, _NT), 0).astype(jnp.float32)

    x = x_ref[0, :, :]
    xa_ref[0:_D, :] = -2.0 * x
    xa_ref[_D:_D + 1, :] = jnp.sum(x * x, axis=0, keepdims=True)
    xa_ref[_D + 1:_DA, :] = jnp.ones((1, _NT), jnp.float32)
    xa = xa_ref[:, :]

    run_min = jnp.full((1, _NT), jnp.inf, jnp.float32)
    run_arg = jnp.full((1, _NT), 0.0, jnp.float32)
    for c in range(_K // _KC):
        raw = lax.dot_general(
            ea_ref[pl.ds(c * _KC, _KC), :], xa, (((1,), (0,)), ((), ())),
            preferred_element_type=jnp.float32)
        cmin = jnp.min(raw, axis=0, keepdims=True)
        carg = jnp.min(jnp.where(raw == cmin, io_ref[:, :], jnp.float32(1e9)),
                       axis=0, keepdims=True)
        if c:
            carg = carg + jnp.float32(c * _KC)
            better = cmin < run_min
            run_arg = jnp.where(better, carg, run_arg)
            run_min = jnp.where(better, cmin, run_min)
        else:
            run_arg, run_min = carg, cmin

    idx_ref[0, :, :] = run_arg.astype(jnp.int32)

    @pl.when(i == 0)
    def _():
        loss_ref[0, 0] = 0.0

    loss_ref[0, 0] += jnp.sum(run_min)

    @pl.when(i == pl.num_programs(0) - 1)
    def _():
        loss_ref[0, 0] = loss_ref[0, 0] * ((1.0 + _COMMIT) / (_N * _D))


def _tc_argmin(x3, emb):
    grid = (_N // _NT,)
    idx, loss, emb_pad = pl.pallas_call(
        _argmin_body,
        grid=grid,
        in_specs=[
            pl.BlockSpec((1, _D, _NT), lambda i: (i, 0, 0)),
            pl.BlockSpec((_K, _D), lambda i: (0, 0)),
        ],
        out_specs=[
            pl.BlockSpec((1, 1, _NT), lambda i: (i, 0, 0)),
            pl.BlockSpec(memory_space=pltpu.MemorySpace.SMEM),
            pl.BlockSpec((_K, _DP), lambda i: (0, 0)),
        ],
        out_shape=[
            jax.ShapeDtypeStruct((grid[0], 1, _NT), jnp.int32),
            jax.ShapeDtypeStruct((1, 1), jnp.float32),
            jax.ShapeDtypeStruct((_K, _DP), jnp.float32),
        ],
        scratch_shapes=[
            pltpu.VMEM((_DA, _NT), jnp.float32),
            pltpu.VMEM((_K, _DA), jnp.float32),
            pltpu.VMEM((_KC, _NT), jnp.float32),
        ],
    )(x3, emb)
    return idx.reshape(_N), loss[0, 0], emb_pad


def _sc_gather(emb_pad, idx):
    info = plsc.get_sparse_core_info()
    nw = info.num_cores * info.num_subcores
    bpw = _N // nw
    mesh = plsc.VectorSubcoreMesh(core_axis_name="c", subcore_axis_name="s")

    @functools.partial(
        pl.kernel, mesh=mesh,
        out_type=jax.ShapeDtypeStruct((_N, _DP), jnp.float32),
        scratch_types=[
            pltpu.VMEM((bpw,), jnp.int32),
            pltpu.VMEM((bpw, _DP), jnp.float32),
            pltpu.SemaphoreType.DMA,
        ],
    )
    def gather(table_hbm, idx_hbm, out_hbm, idx_v, rows_v, sem):
        wid = lax.axis_index("s") * info.num_cores + lax.axis_index("c")
        base = wid * bpw
        pltpu.sync_copy(idx_hbm.at[pl.ds(base, bpw)], idx_v)
        pltpu.async_copy(table_hbm.at[idx_v], rows_v, sem).wait()
        pltpu.sync_copy(rows_v, out_hbm.at[pl.ds(base, bpw)])

    return gather(emb_pad, idx)


def kernel(x, embedding_weight):
    b, c, h, w = x.shape
    x3 = x.reshape(b, c, h * w)
    idx, loss, emb_pad = _tc_argmin(x3, embedding_weight)
    quant_flat = _sc_gather(emb_pad, idx)[:, :_D]
    quantized_out = jnp.transpose(
        quant_flat.reshape(b, h, w, c), (0, 3, 1, 2))
    indices_out = idx.reshape(b, h * w)
    return (loss, quantized_out, indices_out)


# speedup vs baseline: 1.7121x; 1.0064x over previous
import functools

import jax
import jax.numpy as jnp
from jax import lax
from jax.experimental import pallas as pl
from jax.experimental.pallas import tpu as pltpu
from jax.experimental.pallas import tpu_sc as plsc

_K = 8192
_D = 32
_N = 4096
_NT = 1024
_KC = 512
_DA = _D + 2
_DP = 128
_COMMIT = 0.25


def _argmin_body(x_ref, emb_ref, idx_ref, loss_ref, pad_ref,
                 xa_ref, ea_ref, io_ref):
    i = pl.program_id(0)

    @pl.when(i == 0)
    def _():
        e = emb_ref[:, :]
        ea_ref[:, 0:_D] = e
        ea_ref[:, _D:_D + 1] = jnp.ones((_K, 1), jnp.float32)
        ea_ref[:, _D + 1:_DA] = jnp.sum(e * e, axis=1, keepdims=True)
        pad_ref[:, :] = jnp.zeros((_K, _DP), jnp.float32)
        pad_ref[:, 0:_D] = e
        io_ref[:, :] = lax.broadcasted_iota(
            jnp.int32, (_KC, _NT), 0).astype(jnp.float32)

    x = x_ref[0, :, :]
    xa_ref[0:_D, :] = -2.0 * x
    xa_ref[_D:_D + 1, :] = jnp.sum(x * x, axis=0, keepdims=True)
    xa_ref[_D + 1:_DA, :] = jnp.ones((1, _NT), jnp.float32)
    xa = xa_ref[:, :]

    run_min = jnp.full((1, _NT), jnp.inf, jnp.float32)
    run_arg = jnp.full((1, _NT), 0.0, jnp.float32)
    for c in range(_K // _KC):
        raw = lax.dot_general(
            ea_ref[pl.ds(c * _KC, _KC), :], xa, (((1,), (0,)), ((), ())),
            preferred_element_type=jnp.float32)
        cmin = jnp.min(raw, axis=0, keepdims=True)
        carg = jnp.min(jnp.where(raw == cmin, io_ref[:, :], jnp.float32(1e9)),
                       axis=0, keepdims=True)
        if c:
            carg = carg + jnp.float32(c * _KC)
            better = cmin < run_min
            run_arg = jnp.where(better, carg, run_arg)
            run_min = jnp.where(better, cmin, run_min)
        else:
            run_arg, run_min = carg, cmin

    idx_ref[0, :, :] = run_arg.astype(jnp.int32)

    @pl.when(i == 0)
    def _():
        loss_ref[0, 0] = 0.0

    loss_ref[0, 0] += jnp.sum(run_min)

    @pl.when(i == pl.num_programs(0) - 1)
    def _():
        loss_ref[0, 0] = loss_ref[0, 0] * ((1.0 + _COMMIT) / (_N * _D))


def _tc_argmin(x3, emb):
    grid = (_N // _NT,)
    idx, loss, emb_pad = pl.pallas_call(
        _argmin_body,
        grid=grid,
        in_specs=[
            pl.BlockSpec((1, _D, _NT), lambda i: (i, 0, 0)),
            pl.BlockSpec((_K, _D), lambda i: (0, 0)),
        ],
        out_specs=[
            pl.BlockSpec((1, 1, _NT), lambda i: (i, 0, 0)),
            pl.BlockSpec(memory_space=pltpu.MemorySpace.SMEM),
            pl.BlockSpec((_K, _DP), lambda i: (0, 0)),
        ],
        out_shape=[
            jax.ShapeDtypeStruct((grid[0], 1, _NT), jnp.int32),
            jax.ShapeDtypeStruct((1, 1), jnp.float32),
            jax.ShapeDtypeStruct((_K, _DP), jnp.float32),
        ],
        scratch_shapes=[
            pltpu.VMEM((_DA, _NT), jnp.float32),
            pltpu.VMEM((_K, _DA), jnp.float32),
            pltpu.VMEM((_KC, _NT), jnp.float32),
        ],
    )(x3, emb)
    return idx.reshape(_N), loss[0, 0], emb_pad


def _sc_gather(emb_pad, idx):
    info = plsc.get_sparse_core_info()
    nw = info.num_cores * info.num_subcores
    bpw = _N // nw
    mesh = plsc.VectorSubcoreMesh(core_axis_name="c", subcore_axis_name="s")

    @functools.partial(
        pl.kernel, mesh=mesh,
        out_type=jax.ShapeDtypeStruct((_N, _DP), jnp.float32),
        scratch_types=[
            pltpu.VMEM((bpw,), jnp.int32),
            pltpu.VMEM((bpw, _DP), jnp.float32),
            pltpu.SemaphoreType.DMA,
        ],
    )
    def gather(table_hbm, idx_hbm, out_hbm, idx_v, rows_v, sem):
        wid = lax.axis_index("s") * info.num_cores + lax.axis_index("c")
        base = wid * bpw
        pltpu.sync_copy(idx_hbm.at[pl.ds(base, bpw)], idx_v)
        pltpu.async_copy(table_hbm.at[idx_v], rows_v, sem).wait()
        pltpu.sync_copy(rows_v, out_hbm.at[pl.ds(base, bpw)])

    return gather(emb_pad, idx)


def kernel(x, embedding_weight):
    b, c, h, w = x.shape
    x3 = x.reshape(b, c, h * w)
    idx, loss, emb_pad = _tc_argmin(x3, embedding_weight)
    quant_flat = _sc_gather(emb_pad, idx)[:, :_D]
    quantized_out = jnp.transpose(
        quant_flat.reshape(b, h, w, c), (0, 3, 1, 2))
    indices_out = idx.reshape(b, h * w)
    return (loss, quantized_out, indices_out)


# speedup vs baseline: 1.7656x; 1.0313x over previous
import functools

import jax
import jax.numpy as jnp
from jax import lax
from jax.experimental import pallas as pl
from jax.experimental.pallas import tpu as pltpu
from jax.experimental.pallas import tpu_sc as plsc

_K = 8192
_D = 32
_N = 4096
_NT = 4096
_KC = 512
_DA = _D + 2
_DP = 128
_COMMIT = 0.25


def _argmin_body(x_ref, emb_ref, idx_ref, loss_ref, pad_ref,
                 xa_ref, ea_ref, io_ref):
    i = pl.program_id(0)

    @pl.when(i == 0)
    def _():
        e = emb_ref[:, :]
        ea_ref[:, 0:_D] = e
        ea_ref[:, _D:_D + 1] = jnp.ones((_K, 1), jnp.float32)
        ea_ref[:, _D + 1:_DA] = jnp.sum(e * e, axis=1, keepdims=True)
        pad_ref[:, :] = jnp.zeros((_K, _DP), jnp.float32)
        pad_ref[:, 0:_D] = e
        io_ref[:, :] = lax.broadcasted_iota(
            jnp.int32, (_KC, _NT), 0).astype(jnp.float32)

    for bb in range(_NT // 1024):
        xb = x_ref[bb, :, :]
        xa_ref[0:_D, pl.ds(bb * 1024, 1024)] = -2.0 * xb
        xa_ref[_D:_D + 1, pl.ds(bb * 1024, 1024)] = jnp.sum(
            xb * xb, axis=0, keepdims=True)
    xa_ref[_D + 1:_DA, :] = jnp.ones((1, _NT), jnp.float32)
    xa = xa_ref[:, :]

    run_min = jnp.full((1, _NT), jnp.inf, jnp.float32)
    run_arg = jnp.full((1, _NT), 0.0, jnp.float32)
    for c in range(_K // _KC):
        raw = lax.dot_general(
            ea_ref[pl.ds(c * _KC, _KC), :], xa, (((1,), (0,)), ((), ())),
            preferred_element_type=jnp.float32)
        cmin = jnp.min(raw, axis=0, keepdims=True)
        carg = jnp.min(jnp.where(raw == cmin, io_ref[:, :], jnp.float32(1e9)),
                       axis=0, keepdims=True)
        if c:
            carg = carg + jnp.float32(c * _KC)
            better = cmin < run_min
            run_arg = jnp.where(better, carg, run_arg)
            run_min = jnp.where(better, cmin, run_min)
        else:
            run_arg, run_min = carg, cmin

    idx_ref[0, :, :] = run_arg.astype(jnp.int32)

    @pl.when(i == 0)
    def _():
        loss_ref[0, 0] = 0.0

    loss_ref[0, 0] += jnp.sum(run_min)

    @pl.when(i == pl.num_programs(0) - 1)
    def _():
        loss_ref[0, 0] = loss_ref[0, 0] * ((1.0 + _COMMIT) / (_N * _D))


def _tc_argmin(x3, emb):
    grid = (_N // _NT,)
    idx, loss, emb_pad = pl.pallas_call(
        _argmin_body,
        grid=grid,
        in_specs=[
            pl.BlockSpec((_NT // 1024, _D, 1024), lambda i: (i, 0, 0)),
            pl.BlockSpec((_K, _D), lambda i: (0, 0)),
        ],
        out_specs=[
            pl.BlockSpec((1, 1, _NT), lambda i: (i, 0, 0)),
            pl.BlockSpec(memory_space=pltpu.MemorySpace.SMEM),
            pl.BlockSpec((_K, _DP), lambda i: (0, 0)),
        ],
        out_shape=[
            jax.ShapeDtypeStruct((grid[0], 1, _NT), jnp.int32),
            jax.ShapeDtypeStruct((1, 1), jnp.float32),
            jax.ShapeDtypeStruct((_K, _DP), jnp.float32),
        ],
        scratch_shapes=[
            pltpu.VMEM((_DA, _NT), jnp.float32),
            pltpu.VMEM((_K, _DA), jnp.float32),
            pltpu.VMEM((_KC, _NT), jnp.float32),
        ],
    )(x3, emb)
    return idx.reshape(_N), loss[0, 0], emb_pad


def _sc_gather(emb_pad, idx):
    info = plsc.get_sparse_core_info()
    nw = info.num_cores * info.num_subcores
    bpw = _N // nw
    mesh = plsc.VectorSubcoreMesh(core_axis_name="c", subcore_axis_name="s")

    @functools.partial(
        pl.kernel, mesh=mesh,
        out_type=jax.ShapeDtypeStruct((_N, _DP), jnp.float32),
        scratch_types=[
            pltpu.VMEM((bpw,), jnp.int32),
            pltpu.VMEM((bpw, _DP), jnp.float32),
            pltpu.SemaphoreType.DMA,
        ],
    )
    def gather(table_hbm, idx_hbm, out_hbm, idx_v, rows_v, sem):
        wid = lax.axis_index("s") * info.num_cores + lax.axis_index("c")
        base = wid * bpw
        pltpu.sync_copy(idx_hbm.at[pl.ds(base, bpw)], idx_v)
        pltpu.async_copy(table_hbm.at[idx_v], rows_v, sem).wait()
        pltpu.sync_copy(rows_v, out_hbm.at[pl.ds(base, bpw)])

    return gather(emb_pad, idx)


def kernel(x, embedding_weight):
    b, c, h, w = x.shape
    x3 = x.reshape(b, c, h * w)
    idx, loss, emb_pad = _tc_argmin(x3, embedding_weight)
    quant_flat = _sc_gather(emb_pad, idx)[:, :_D]
    quantized_out = jnp.transpose(
        quant_flat.reshape(b, h, w, c), (0, 3, 1, 2))
    indices_out = idx.reshape(b, h * w)
    return (loss, quantized_out, indices_out)


# speedup vs baseline: 1.7888x; 1.0131x over previous
import functools

import jax
import jax.numpy as jnp
from jax import lax
from jax.experimental import pallas as pl
from jax.experimental.pallas import tpu as pltpu
from jax.experimental.pallas import tpu_sc as plsc

_K = 8192
_D = 32
_N = 4096
_NT = 4096
_KC = 256
_DA = _D + 2
_DP = 128
_COMMIT = 0.25


def _argmin_body(x_ref, emb_ref, idx_ref, loss_ref, pad_ref,
                 xa_ref, ea_ref, io_ref):
    i = pl.program_id(0)

    @pl.when(i == 0)
    def _():
        e = emb_ref[:, :]
        ea_ref[:, 0:_D] = e
        ea_ref[:, _D:_D + 1] = jnp.ones((_K, 1), jnp.float32)
        ea_ref[:, _D + 1:_DA] = jnp.sum(e * e, axis=1, keepdims=True)
        pad_ref[:, :] = jnp.zeros((_K, _DP), jnp.float32)
        pad_ref[:, 0:_D] = e
        io_ref[:, :] = lax.broadcasted_iota(
            jnp.int32, (_KC, _NT), 0).astype(jnp.float32)

    for bb in range(_NT // 1024):
        xb = x_ref[bb, :, :]
        xa_ref[0:_D, pl.ds(bb * 1024, 1024)] = -2.0 * xb
        xa_ref[_D:_D + 1, pl.ds(bb * 1024, 1024)] = jnp.sum(
            xb * xb, axis=0, keepdims=True)
    xa_ref[_D + 1:_DA, :] = jnp.ones((1, _NT), jnp.float32)
    xa = xa_ref[:, :]

    run_min = jnp.full((1, _NT), jnp.inf, jnp.float32)
    run_arg = jnp.full((1, _NT), 0.0, jnp.float32)
    for c in range(_K // _KC):
        raw = lax.dot_general(
            ea_ref[pl.ds(c * _KC, _KC), :], xa, (((1,), (0,)), ((), ())),
            preferred_element_type=jnp.float32)
        cmin = jnp.min(raw, axis=0, keepdims=True)
        carg = jnp.min(jnp.where(raw == cmin, io_ref[:, :], jnp.float32(1e9)),
                       axis=0, keepdims=True)
        if c:
            carg = carg + jnp.float32(c * _KC)
            better = cmin < run_min
            run_arg = jnp.where(better, carg, run_arg)
            run_min = jnp.where(better, cmin, run_min)
        else:
            run_arg, run_min = carg, cmin

    idx_ref[0, :, :] = run_arg.astype(jnp.int32)

    @pl.when(i == 0)
    def _():
        loss_ref[0, 0] = 0.0

    loss_ref[0, 0] += jnp.sum(run_min)

    @pl.when(i == pl.num_programs(0) - 1)
    def _():
        loss_ref[0, 0] = loss_ref[0, 0] * ((1.0 + _COMMIT) / (_N * _D))


def _tc_argmin(x3, emb):
    grid = (_N // _NT,)
    idx, loss, emb_pad = pl.pallas_call(
        _argmin_body,
        grid=grid,
        in_specs=[
            pl.BlockSpec((_NT // 1024, _D, 1024), lambda i: (i, 0, 0)),
            pl.BlockSpec((_K, _D), lambda i: (0, 0)),
        ],
        out_specs=[
            pl.BlockSpec((1, 1, _NT), lambda i: (i, 0, 0)),
            pl.BlockSpec(memory_space=pltpu.MemorySpace.SMEM),
            pl.BlockSpec((_K, _DP), lambda i: (0, 0)),
        ],
        out_shape=[
            jax.ShapeDtypeStruct((grid[0], 1, _NT), jnp.int32),
            jax.ShapeDtypeStruct((1, 1), jnp.float32),
            jax.ShapeDtypeStruct((_K, _DP), jnp.float32),
        ],
        scratch_shapes=[
            pltpu.VMEM((_DA, _NT), jnp.float32),
            pltpu.VMEM((_K, _DA), jnp.float32),
            pltpu.VMEM((_KC, _NT), jnp.float32),
        ],
    )(x3, emb)
    return idx.reshape(_N), loss[0, 0], emb_pad


def _sc_gather(emb_pad, idx):
    info = plsc.get_sparse_core_info()
    nw = info.num_cores * info.num_subcores
    bpw = _N // nw
    mesh = plsc.VectorSubcoreMesh(core_axis_name="c", subcore_axis_name="s")

    @functools.partial(
        pl.kernel, mesh=mesh,
        out_type=jax.ShapeDtypeStruct((_N, _DP), jnp.float32),
        scratch_types=[
            pltpu.VMEM((bpw,), jnp.int32),
            pltpu.VMEM((bpw, _DP), jnp.float32),
            pltpu.SemaphoreType.DMA,
        ],
    )
    def gather(table_hbm, idx_hbm, out_hbm, idx_v, rows_v, sem):
        wid = lax.axis_index("s") * info.num_cores + lax.axis_index("c")
        base = wid * bpw
        pltpu.sync_copy(idx_hbm.at[pl.ds(base, bpw)], idx_v)
        pltpu.async_copy(table_hbm.at[idx_v], rows_v, sem).wait()
        pltpu.sync_copy(rows_v, out_hbm.at[pl.ds(base, bpw)])

    return gather(emb_pad, idx)


def kernel(x, embedding_weight):
    b, c, h, w = x.shape
    x3 = x.reshape(b, c, h * w)
    idx, loss, emb_pad = _tc_argmin(x3, embedding_weight)
    quant_flat = _sc_gather(emb_pad, idx)[:, :_D]
    quantized_out = jnp.transpose(
        quant_flat.reshape(b, h, w, c), (0, 3, 1, 2))
    indices_out = idx.reshape(b, h * w)
    return (loss, quantized_out, indices_out)


# speedup vs baseline: 1.9700x; 1.1013x over previous
import functools

import jax
import jax.numpy as jnp
from jax import lax
from jax.experimental import pallas as pl
from jax.experimental.pallas import tpu as pltpu
from jax.experimental.pallas import tpu_sc as plsc

_K = 8192
_D = 32
_N = 4096
_NT = 4096
_KC = 256
_DA = _D + 2
_DP = 128
_COMMIT = 0.25


def _argmin_body(x_ref, emb_ref, idx_ref, loss_ref, pad_ref,
                 xa_ref, ea_ref, io_ref):
    i = pl.program_id(0)

    @pl.when(i == 0)
    def _():
        e = emb_ref[:, :]
        ea_ref[:, 0:_D] = e
        ea_ref[:, _D:_D + 1] = jnp.ones((_K, 1), jnp.float32)
        ea_ref[:, _D + 1:_DA] = jnp.sum(e * e, axis=1, keepdims=True)
        pad_ref[:, :] = jnp.zeros((_K, _DP), jnp.float32)
        pad_ref[:, 0:_D] = e
        io_ref[:, :] = lax.broadcasted_iota(
            jnp.int32, (_KC, _NT), 0).astype(jnp.float32)

    for bb in range(_NT // 1024):
        xb = x_ref[bb, :, :]
        xa_ref[0:_D, pl.ds(bb * 1024, 1024)] = -2.0 * xb
        xa_ref[_D:_D + 1, pl.ds(bb * 1024, 1024)] = jnp.sum(
            xb * xb, axis=0, keepdims=True)
    xa_ref[_D + 1:_DA, :] = jnp.ones((1, _NT), jnp.float32)
    xa = xa_ref[:, :]

    run_min = jnp.full((1, _NT), jnp.inf, jnp.float32)
    run_arg = jnp.full((1, _NT), 0, jnp.int32)
    for c in range(_K // _KC):
        raw = lax.dot_general(
            ea_ref[pl.ds(c * _KC, _KC), :], xa, (((1,), (0,)), ((), ())),
            preferred_element_type=jnp.float32)
        cmin = jnp.min(raw, axis=0, keepdims=True)
        carg = jnp.argmin(raw, axis=0).reshape(1, _NT)
        if c:
            carg = carg + jnp.int32(c * _KC)
            better = cmin < run_min
            run_arg = jnp.where(better, carg, run_arg)
            run_min = jnp.where(better, cmin, run_min)
        else:
            run_arg, run_min = carg, cmin

    idx_ref[0, :, :] = run_arg

    @pl.when(i == 0)
    def _():
        loss_ref[0, 0] = 0.0

    loss_ref[0, 0] += jnp.sum(run_min)

    @pl.when(i == pl.num_programs(0) - 1)
    def _():
        loss_ref[0, 0] = loss_ref[0, 0] * ((1.0 + _COMMIT) / (_N * _D))


def _tc_argmin(x3, emb):
    grid = (_N // _NT,)
    idx, loss, emb_pad = pl.pallas_call(
        _argmin_body,
        grid=grid,
        in_specs=[
            pl.BlockSpec((_NT // 1024, _D, 1024), lambda i: (i, 0, 0)),
            pl.BlockSpec((_K, _D), lambda i: (0, 0)),
        ],
        out_specs=[
            pl.BlockSpec((1, 1, _NT), lambda i: (i, 0, 0)),
            pl.BlockSpec(memory_space=pltpu.MemorySpace.SMEM),
            pl.BlockSpec((_K, _DP), lambda i: (0, 0)),
        ],
        out_shape=[
            jax.ShapeDtypeStruct((grid[0], 1, _NT), jnp.int32),
            jax.ShapeDtypeStruct((1, 1), jnp.float32),
            jax.ShapeDtypeStruct((_K, _DP), jnp.float32),
        ],
        scratch_shapes=[
            pltpu.VMEM((_DA, _NT), jnp.float32),
            pltpu.VMEM((_K, _DA), jnp.float32),
            pltpu.VMEM((_KC, _NT), jnp.float32),
        ],
    )(x3, emb)
    return idx.reshape(_N), loss[0, 0], emb_pad


def _sc_gather(emb_pad, idx):
    info = plsc.get_sparse_core_info()
    nw = info.num_cores * info.num_subcores
    bpw = _N // nw
    mesh = plsc.VectorSubcoreMesh(core_axis_name="c", subcore_axis_name="s")

    @functools.partial(
        pl.kernel, mesh=mesh,
        out_type=jax.ShapeDtypeStruct((_N, _DP), jnp.float32),
        scratch_types=[
            pltpu.VMEM((bpw,), jnp.int32),
            pltpu.VMEM((bpw, _DP), jnp.float32),
            pltpu.SemaphoreType.DMA,
        ],
    )
    def gather(table_hbm, idx_hbm, out_hbm, idx_v, rows_v, sem):
        wid = lax.axis_index("s") * info.num_cores + lax.axis_index("c")
        base = wid * bpw
        pltpu.sync_copy(idx_hbm.at[pl.ds(base, bpw)], idx_v)
        pltpu.async_copy(table_hbm.at[idx_v], rows_v, sem).wait()
        pltpu.sync_copy(rows_v, out_hbm.at[pl.ds(base, bpw)])

    return gather(emb_pad, idx)


def kernel(x, embedding_weight):
    b, c, h, w = x.shape
    x3 = x.reshape(b, c, h * w)
    idx, loss, emb_pad = _tc_argmin(x3, embedding_weight)
    quant_flat = _sc_gather(emb_pad, idx)[:, :_D]
    quantized_out = jnp.transpose(
        quant_flat.reshape(b, h, w, c), (0, 3, 1, 2))
    indices_out = idx.reshape(b, h * w)
    return (loss, quantized_out, indices_out)
